# baseline (speedup 1.0000x reference)
import functools

import jax
import jax.numpy as jnp
from jax import lax
from jax.experimental import pallas as pl
from jax.experimental.pallas import tpu as pltpu
from jax.experimental.pallas import tpu_sc as plsc

N = 10000
E = 320000
D = 128
NC = 2
NS = 16
NT = NC * NS
EPT = E // NT
K = 80
NCH = EPT // K
B = 5
NBLK = NCH // B
GRP = 6
NP = 10240
RPT = NP // NS
L = 16


def _prep_body(x_ref, w_ref, as_ref, ad_ref, h_ref, asum_ref, adsum_ref):
    h = jnp.dot(x_ref[...], w_ref[...], preferred_element_type=jnp.float32)
    h_ref[...] = h
    asum_ref[...] = jnp.sum(h * as_ref[...], axis=1)
    adsum_ref[...] = jnp.sum(h * ad_ref[...], axis=1)


_prep = pl.pallas_call(
    _prep_body,
    out_shape=[
        jax.ShapeDtypeStruct((N, D), jnp.float32),
        jax.ShapeDtypeStruct((N,), jnp.float32),
        jax.ShapeDtypeStruct((N,), jnp.float32),
    ],
)


def _finish_body(outp_ref, denp_ref, g_ref, b_ref, o_ref):
    s = outp_ref[0, :N, :] + outp_ref[1, :N, :]
    den = jnp.sum(denp_ref[...], axis=0)[:N]
    q = s / (den[:, None] + 1e-16)
    mean = jnp.mean(q, axis=0)
    qc = q - mean[None, :]
    var = jnp.mean(qc * qc, axis=0)
    scale = g_ref[...] / jnp.sqrt(var + 1e-5)
    o_ref[...] = jnp.maximum(qc * scale[None, :] + b_ref[...][None, :], 0.0)


_finish = pl.pallas_call(
    _finish_body,
    out_shape=jax.ShapeDtypeStruct((N, D), jnp.float32),
)


def _edge_body(asrc_hbm, adst_hbm, srcf_hbm, dstf_hbm,
               eexp_hbm, denp_hbm,
               asrc_v, adst_v, src_v, dst_v, eexp_v, denom_v):
    cid = lax.axis_index("c")
    sid = lax.axis_index("s")
    wid = sid * NC + cid

    zeros16 = jnp.zeros((L,), jnp.float32)

    @pl.loop(0, NP // L)
    def _zero_denom(i):
        denom_v[pl.ds(i * L, L)] = zeros16

    pltpu.sync_copy(asrc_hbm, asrc_v)
    pltpu.sync_copy(adst_hbm, adst_v)
    pltpu.sync_copy(srcf_hbm.at[pl.ds(wid * EPT, EPT)], src_v)
    pltpu.sync_copy(dstf_hbm.at[pl.ds(wid * EPT, EPT)], dst_v)

    @plsc.parallel_loop(0, NCH, unroll=4)
    def _edge_chunk(j):
        for v in range(K // L):
            o = j * K + v * L
            sidx = src_v[pl.ds(o, L)]
            didx = dst_v[pl.ds(o, L)]
            av = plsc.load_gather(asrc_v, [sidx])
            bv = plsc.load_gather(adst_v, [didx])
            e = av + bv
            e = jnp.where(e >= 0.0, e, 0.2 * e)
            ex = jnp.exp(e)
            eexp_v[pl.ds(o, L)] = ex
            plsc.addupdate_scatter(denom_v, [didx], ex)

    pltpu.sync_copy(eexp_v, eexp_hbm.at[pl.ds(wid * EPT, EPT)])
    pltpu.sync_copy(denom_v, denp_hbm.at[wid])


_edge_sc = functools.partial(
    pl.kernel,
    out_type=[
        jax.ShapeDtypeStruct((E,), jnp.float32),
        jax.ShapeDtypeStruct((NT, NP), jnp.float32),
    ],
    mesh=plsc.VectorSubcoreMesh(core_axis_name="c", subcore_axis_name="s"),
    compiler_params=pltpu.CompilerParams(needs_layout_passes=False),
    scratch_types=[
        pltpu.VMEM((N,), jnp.float32),
        pltpu.VMEM((N,), jnp.float32),
        pltpu.VMEM((EPT,), jnp.int32),
        pltpu.VMEM((EPT,), jnp.int32),
        pltpu.VMEM((EPT,), jnp.float32),
        pltpu.VMEM((NP,), jnp.float32),
    ],
)(_edge_body)


def _rows_body(h_hbm, eexpf_hbm, srcf_hbm, dst4_hbm,
               outp_hbm,
               src_blk0, src_blk1, dst_blk0, dst_blk1, eexp_blk0, eexp_blk1,
               rows_0, rows_1, rows_2,
               acc_sh,
               gsem0, gsem1, gsem2, ssem0, ssem1, ssem2, stsem0, stsem1):
    cid = lax.axis_index("c")
    sid = lax.axis_index("s")
    wid = sid * NC + cid
    src_blks = (src_blk0, src_blk1)
    dst_blks = (dst_blk0, dst_blk1)
    eexp_blks = (eexp_blk0, eexp_blk1)
    rowss = (rows_0, rows_1, rows_2)
    gsem = (gsem0, gsem1, gsem2)
    ssem = (ssem0, ssem1, ssem2)
    stsem = (stsem0, stsem1)

    zeros16 = jnp.zeros((L,), jnp.float32)

    @pl.loop(0, K)
    def _zero_rows(r):
        for c in range(D // L):
            rows_0[r, pl.ds(c * L, L)] = zeros16

    base_r = sid * RPT
    for t in range(RPT // K):
        pltpu.sync_copy(rows_0, acc_sh.at[pl.ds(base_r + t * K, K)])

    pltpu.sync_copy(srcf_hbm.at[pl.ds(wid * EPT, B * K)], src_blk0)
    pltpu.sync_copy(dst4_hbm.at[wid, 0], dst_blk0)
    pltpu.sync_copy(eexpf_hbm.at[pl.ds(wid * EPT, B * K)], eexp_blk0)

    plsc.subcore_barrier()

    pltpu.async_copy(h_hbm.at[src_blk0.at[pl.ds(0, K)]], rows_0, gsem[0])

    def wait_gather(p):
        pltpu.make_async_copy(h_hbm.at[src_blk0.at[pl.ds(0, K)]],
                              rowss[p], gsem[p]).wait()

    def wait_scatter(p):
        pltpu.make_async_copy(rowss[p],
                              acc_sh.at[dst_blk0.at[0]], ssem[p]).wait()

    def stage_issue(bi_next, nb):
        pltpu.async_copy(
            srcf_hbm.at[pl.ds(wid * EPT + bi_next * (B * K), B * K)],
            src_blks[nb], stsem[nb])
        pltpu.async_copy(dst4_hbm.at[wid, bi_next], dst_blks[nb], stsem[nb])
        pltpu.async_copy(
            eexpf_hbm.at[pl.ds(wid * EPT + bi_next * (B * K), B * K)],
            eexp_blks[nb], stsem[nb])

    def wait_stage(nb):
        pltpu.make_async_copy(srcf_hbm.at[pl.ds(0, B * K)], src_blks[nb],
                              stsem[nb]).wait()
        pltpu.make_async_copy(dst4_hbm.at[wid, 0], dst_blks[nb],
                              stsem[nb]).wait()
        pltpu.make_async_copy(eexpf_hbm.at[pl.ds(0, B * K)], eexp_blks[nb],
                              stsem[nb]).wait()

    def chunk(pb, b, p, *, skip_scatter_wait=False, stage_bi=None,
              next_rc=None):
        q = (p + 1) % 3
        if stage_bi is not None:
            stage_issue(stage_bi, 1 - pb)
        if not skip_scatter_wait:
            wait_scatter(q)
        if next_rc is not None:
            npb, nb = next_rc
            if b == B - 1:
                wait_stage(npb)
            pltpu.async_copy(h_hbm.at[src_blks[npb].at[pl.ds(nb * K, K)]],
                             rowss[q], gsem[q])
        wait_gather(p)

        rows_p = rowss[p]
        eexp_pb = eexp_blks[pb]

        @plsc.parallel_loop(0, K, unroll=4)
        def _scale_row(r):
            rr = jnp.full((L,), b * K + r, dtype=jnp.int32)
            w = plsc.load_gather(eexp_pb, [rr])
            for c in range(D // L):
                rows_p[r, pl.ds(c * L, L)] = rows_p[r, pl.ds(c * L, L)] * w

        pltpu.async_copy(rows_p, acc_sh.at[dst_blks[pb].at[b]],
                         ssem[p], add=True)

    def group(gbase, *, first=False, last=False):
        nblk_here = 1 if last else GRP
        for bb in range(nblk_here):
            pb = bb % 2
            for b in range(B):
                t_in_group = bb * B + b
                chunk(
                    pb, b, t_in_group % 3,
                    skip_scatter_wait=first and t_in_group < 2,
                    stage_bi=(gbase + bb + 1) if (b == 2 and not last) else None,
                    next_rc=None if (last and b == B - 1) else
                            ((pb, b + 1) if b < B - 1 else (1 - pb, 0)),
                )

    group(jnp.int32(0), first=True)

    @pl.loop(1, (NBLK - 1) // GRP)
    def _grp(g):
        group(g * GRP)

    group(jnp.int32(NBLK - 1), last=True)

    wait_scatter((NCH - 2) % 3)
    wait_scatter((NCH - 1) % 3)
    plsc.subcore_barrier()
    pltpu.sync_copy(acc_sh.at[pl.ds(base_r, RPT)],
                    outp_hbm.at[cid, pl.ds(base_r, RPT)])


_rows_sc = functools.partial(
    pl.kernel,
    out_type=jax.ShapeDtypeStruct((NC, NP, D), jnp.float32),
    mesh=plsc.VectorSubcoreMesh(core_axis_name="c", subcore_axis_name="s"),
    compiler_params=pltpu.CompilerParams(needs_layout_passes=False),
    scratch_types=[
        pltpu.VMEM((B * K,), jnp.int32),
        pltpu.VMEM((B * K,), jnp.int32),
        pltpu.VMEM((B, K), jnp.int32),
        pltpu.VMEM((B, K), jnp.int32),
        pltpu.VMEM((B * K,), jnp.float32),
        pltpu.VMEM((B * K,), jnp.float32),
        pltpu.VMEM((K, D), jnp.float32),
        pltpu.VMEM((K, D), jnp.float32),
        pltpu.VMEM((K, D), jnp.float32),
        pltpu.VMEM_SHARED((NP, D), jnp.float32),
        pltpu.SemaphoreType.DMA,
        pltpu.SemaphoreType.DMA,
        pltpu.SemaphoreType.DMA,
        pltpu.SemaphoreType.DMA,
        pltpu.SemaphoreType.DMA,
        pltpu.SemaphoreType.DMA,
        pltpu.SemaphoreType.DMA,
        pltpu.SemaphoreType.DMA,
    ],
)(_rows_body)


def kernel(x, adj, W, att_src, att_dst, bn_gamma, bn_beta):
    src_f = adj[0].astype(jnp.int32)
    dst_f = adj[1].astype(jnp.int32)
    dst4 = dst_f.reshape(NT, NBLK, B, K)
    h, a_src, a_dst = _prep(x, W, att_src, att_dst)
    eexp_f, denp = _edge_sc(a_src, a_dst, src_f, dst_f)
    outp = _rows_sc(h, eexp_f, src_f, dst4)
    return _finish(outp, denp, bn_gamma, bn_beta)


# speedup vs baseline: 1.0199x; 1.0199x over previous
import functools

import jax
import jax.numpy as jnp
from jax import lax
from jax.experimental import pallas as pl
from jax.experimental.pallas import tpu as pltpu
from jax.experimental.pallas import tpu_sc as plsc

N = 10000
E = 320000
D = 128
NC = 2
NS = 16
NT = NC * NS
EPT = E // NT
K = 80
NCH = EPT // K
B = 5
NBLK = NCH // B
GRP = 4
NP = 10240
RPT = NP // NS
L = 16


def _prep_body(x_ref, w_ref, as_ref, ad_ref, h_ref, asum_ref, adsum_ref):
    h = jnp.dot(x_ref[...], w_ref[...], preferred_element_type=jnp.float32)
    h_ref[...] = h
    asum_ref[...] = jnp.sum(h * as_ref[...], axis=1)
    adsum_ref[...] = jnp.sum(h * ad_ref[...], axis=1)


_prep = pl.pallas_call(
    _prep_body,
    out_shape=[
        jax.ShapeDtypeStruct((N, D), jnp.float32),
        jax.ShapeDtypeStruct((N,), jnp.float32),
        jax.ShapeDtypeStruct((N,), jnp.float32),
    ],
)


def _finish_body(outp_ref, denp_ref, g_ref, b_ref, o_ref):
    s = outp_ref[0, :N, :] + outp_ref[1, :N, :]
    den = jnp.sum(denp_ref[...], axis=0)[:N]
    q = s / (den[:, None] + 1e-16)
    mean = jnp.mean(q, axis=0)
    qc = q - mean[None, :]
    var = jnp.mean(qc * qc, axis=0)
    scale = g_ref[...] / jnp.sqrt(var + 1e-5)
    o_ref[...] = jnp.maximum(qc * scale[None, :] + b_ref[...][None, :], 0.0)


_finish = pl.pallas_call(
    _finish_body,
    out_shape=jax.ShapeDtypeStruct((N, D), jnp.float32),
)


def _edge_body(asrc_hbm, adst_hbm, srcf_hbm, dstf_hbm,
               eexp_hbm, denp_hbm,
               asrc_v, adst_v, src_v, dst_v, eexp_v, denom_v):
    cid = lax.axis_index("c")
    sid = lax.axis_index("s")
    wid = sid * NC + cid

    zeros16 = jnp.zeros((L,), jnp.float32)

    @pl.loop(0, NP // L)
    def _zero_denom(i):
        denom_v[pl.ds(i * L, L)] = zeros16

    pltpu.sync_copy(asrc_hbm, asrc_v)
    pltpu.sync_copy(adst_hbm, adst_v)
    pltpu.sync_copy(srcf_hbm.at[pl.ds(wid * EPT, EPT)], src_v)
    pltpu.sync_copy(dstf_hbm.at[pl.ds(wid * EPT, EPT)], dst_v)

    @plsc.parallel_loop(0, NCH, unroll=2)
    def _edge_chunk(j):
        for v in range(K // L):
            o = j * K + v * L
            sidx = src_v[pl.ds(o, L)]
            didx = dst_v[pl.ds(o, L)]
            av = plsc.load_gather(asrc_v, [sidx])
            bv = plsc.load_gather(adst_v, [didx])
            e = av + bv
            e = jnp.where(e >= 0.0, e, 0.2 * e)
            ex = jnp.exp(e)
            eexp_v[pl.ds(o, L)] = ex
            plsc.addupdate_scatter(denom_v, [didx], ex)

    pltpu.sync_copy(eexp_v, eexp_hbm.at[pl.ds(wid * EPT, EPT)])
    pltpu.sync_copy(denom_v, denp_hbm.at[wid])


_edge_sc = functools.partial(
    pl.kernel,
    out_type=[
        jax.ShapeDtypeStruct((E,), jnp.float32),
        jax.ShapeDtypeStruct((NT, NP), jnp.float32),
    ],
    mesh=plsc.VectorSubcoreMesh(core_axis_name="c", subcore_axis_name="s"),
    compiler_params=pltpu.CompilerParams(needs_layout_passes=False),
    scratch_types=[
        pltpu.VMEM((N,), jnp.float32),
        pltpu.VMEM((N,), jnp.float32),
        pltpu.VMEM((EPT,), jnp.int32),
        pltpu.VMEM((EPT,), jnp.int32),
        pltpu.VMEM((EPT,), jnp.float32),
        pltpu.VMEM((NP,), jnp.float32),
    ],
)(_edge_body)


def _rows_body(h_hbm, eexpf_hbm, srcf_hbm, dst4_hbm,
               outp_hbm,
               src_blk0, src_blk1, dst_blk0, dst_blk1, eexp_blk0, eexp_blk1,
               rows_0, rows_1, rows_2, rows_3,
               acc_sh,
               gsem0, gsem1, gsem2, gsem3, ssem0, ssem1, ssem2, ssem3,
               stsem0, stsem1):
    cid = lax.axis_index("c")
    sid = lax.axis_index("s")
    wid = sid * NC + cid
    src_blks = (src_blk0, src_blk1)
    dst_blks = (dst_blk0, dst_blk1)
    eexp_blks = (eexp_blk0, eexp_blk1)
    rowss = (rows_0, rows_1, rows_2, rows_3)
    gsem = (gsem0, gsem1, gsem2, gsem3)
    ssem = (ssem0, ssem1, ssem2, ssem3)
    stsem = (stsem0, stsem1)

    zeros16 = jnp.zeros((L,), jnp.float32)

    @pl.loop(0, K)
    def _zero_rows(r):
        for c in range(D // L):
            rows_0[r, pl.ds(c * L, L)] = zeros16

    base_r = sid * RPT
    for t in range(RPT // K):
        pltpu.sync_copy(rows_0, acc_sh.at[pl.ds(base_r + t * K, K)])

    pltpu.sync_copy(srcf_hbm.at[pl.ds(wid * EPT, B * K)], src_blk0)
    pltpu.sync_copy(dst4_hbm.at[wid, 0], dst_blk0)
    pltpu.sync_copy(eexpf_hbm.at[pl.ds(wid * EPT, B * K)], eexp_blk0)

    plsc.subcore_barrier()

    pltpu.async_copy(h_hbm.at[src_blk0.at[pl.ds(0, K)]], rows_0, gsem[0])

    def wait_gather(p):
        pltpu.make_async_copy(h_hbm.at[src_blk0.at[pl.ds(0, K)]],
                              rowss[p], gsem[p]).wait()

    def wait_scatter(p):
        pltpu.make_async_copy(rowss[p],
                              acc_sh.at[dst_blk0.at[0]], ssem[p]).wait()

    def stage_issue(bi_next, nb):
        pltpu.async_copy(
            srcf_hbm.at[pl.ds(wid * EPT + bi_next * (B * K), B * K)],
            src_blks[nb], stsem[nb])
        pltpu.async_copy(dst4_hbm.at[wid, bi_next], dst_blks[nb], stsem[nb])
        pltpu.async_copy(
            eexpf_hbm.at[pl.ds(wid * EPT + bi_next * (B * K), B * K)],
            eexp_blks[nb], stsem[nb])

    def wait_stage(nb):
        pltpu.make_async_copy(srcf_hbm.at[pl.ds(0, B * K)], src_blks[nb],
                              stsem[nb]).wait()
        pltpu.make_async_copy(dst4_hbm.at[wid, 0], dst_blks[nb],
                              stsem[nb]).wait()
        pltpu.make_async_copy(eexpf_hbm.at[pl.ds(0, B * K)], eexp_blks[nb],
                              stsem[nb]).wait()

    def chunk(pb, b, p, *, skip_scatter_wait=False, stage_bi=None,
              next_rc=None):
        q = (p + 1) % 4
        if stage_bi is not None:
            stage_issue(stage_bi, 1 - pb)
        if not skip_scatter_wait:
            wait_scatter(q)
        if next_rc is not None:
            npb, nb = next_rc
            if b == B - 1:
                wait_stage(npb)
            pltpu.async_copy(h_hbm.at[src_blks[npb].at[pl.ds(nb * K, K)]],
                             rowss[q], gsem[q])
        wait_gather(p)

        rows_p = rowss[p]
        eexp_pb = eexp_blks[pb]

        @plsc.parallel_loop(0, K, unroll=4)
        def _scale_row(r):
            rr = jnp.full((L,), b * K + r, dtype=jnp.int32)
            w = plsc.load_gather(eexp_pb, [rr])
            for c in range(D // L):
                rows_p[r, pl.ds(c * L, L)] = rows_p[r, pl.ds(c * L, L)] * w

        pltpu.async_copy(rows_p, acc_sh.at[dst_blks[pb].at[b]],
                         ssem[p], add=True)

    def group(gbase, *, first=False, last=False):
        nblk_here = 1 if last else GRP
        for bb in range(nblk_here):
            pb = bb % 2
            for b in range(B):
                t_in_group = bb * B + b
                chunk(
                    pb, b, t_in_group % 4,
                    skip_scatter_wait=first and t_in_group < 3,
                    stage_bi=(gbase + bb + 1) if (b == 2 and not last) else None,
                    next_rc=None if (last and b == B - 1) else
                            ((pb, b + 1) if b < B - 1 else (1 - pb, 0)),
                )

    group(jnp.int32(0), first=True)

    @pl.loop(1, (NBLK - 1) // GRP)
    def _grp(g):
        group(g * GRP)

    group(jnp.int32(NBLK - 1), last=True)

    wait_scatter((NCH - 3) % 4)
    wait_scatter((NCH - 2) % 4)
    wait_scatter((NCH - 1) % 4)
    plsc.subcore_barrier()
    pltpu.sync_copy(acc_sh.at[pl.ds(base_r, RPT)],
                    outp_hbm.at[cid, pl.ds(base_r, RPT)])


_rows_sc = functools.partial(
    pl.kernel,
    out_type=jax.ShapeDtypeStruct((NC, NP, D), jnp.float32),
    mesh=plsc.VectorSubcoreMesh(core_axis_name="c", subcore_axis_name="s"),
    compiler_params=pltpu.CompilerParams(needs_layout_passes=False),
    scratch_types=[
        pltpu.VMEM((B * K,), jnp.int32),
        pltpu.VMEM((B * K,), jnp.int32),
        pltpu.VMEM((B, K), jnp.int32),
        pltpu.VMEM((B, K), jnp.int32),
        pltpu.VMEM((B * K,), jnp.float32),
        pltpu.VMEM((B * K,), jnp.float32),
        pltpu.VMEM((K, D), jnp.float32),
        pltpu.VMEM((K, D), jnp.float32),
        pltpu.VMEM((K, D), jnp.float32),
        pltpu.VMEM((K, D), jnp.float32),
        pltpu.VMEM_SHARED((NP, D), jnp.float32),
        pltpu.SemaphoreType.DMA,
        pltpu.SemaphoreType.DMA,
        pltpu.SemaphoreType.DMA,
        pltpu.SemaphoreType.DMA,
        pltpu.SemaphoreType.DMA,
        pltpu.SemaphoreType.DMA,
        pltpu.SemaphoreType.DMA,
        pltpu.SemaphoreType.DMA,
        pltpu.SemaphoreType.DMA,
        pltpu.SemaphoreType.DMA,
    ],
)(_rows_body)


def kernel(x, adj, W, att_src, att_dst, bn_gamma, bn_beta):
    src_f = adj[0].astype(jnp.int32)
    dst_f = adj[1].astype(jnp.int32)
    dst4 = dst_f.reshape(NT, NBLK, B, K)
    h, a_src, a_dst = _prep(x, W, att_src, att_dst)
    eexp_f, denp = _edge_sc(a_src, a_dst, src_f, dst_f)
    outp = _rows_sc(h, eexp_f, src_f, dst4)
    return _finish(outp, denp, bn_gamma, bn_beta)


# speedup vs baseline: 1.0230x; 1.0030x over previous
import functools

import jax
import jax.numpy as jnp
from jax import lax
from jax.experimental import pallas as pl
from jax.experimental.pallas import tpu as pltpu
from jax.experimental.pallas import tpu_sc as plsc

N = 10000
E = 320000
D = 128
NC = 2
NS = 16
NT = NC * NS
EPT = E // NT
K = 80
NCH = EPT // K
B = 5
NBLK = NCH // B
GRP = 4
NP = 10240
RPT = NP // NS
L = 16


def _prep_body(x_ref, w_ref, as_ref, ad_ref, h_ref, asum_ref, adsum_ref):
    h = jnp.dot(x_ref[...], w_ref[...], preferred_element_type=jnp.float32)
    h_ref[...] = h
    asum_ref[...] = jnp.sum(h * as_ref[...], axis=1)
    adsum_ref[...] = jnp.sum(h * ad_ref[...], axis=1)


_prep = pl.pallas_call(
    _prep_body,
    out_shape=[
        jax.ShapeDtypeStruct((N, D), jnp.float32),
        jax.ShapeDtypeStruct((N,), jnp.float32),
        jax.ShapeDtypeStruct((N,), jnp.float32),
    ],
)


def _finish_body(outp_ref, denp_ref, g_ref, b_ref, o_ref):
    s = outp_ref[0, :N, :] + outp_ref[1, :N, :]
    den = jnp.sum(denp_ref[...], axis=0)[:N]
    q = s / (den[:, None] + 1e-16)
    mean = jnp.mean(q, axis=0)
    qc = q - mean[None, :]
    var = jnp.mean(qc * qc, axis=0)
    scale = g_ref[...] / jnp.sqrt(var + 1e-5)
    o_ref[...] = jnp.maximum(qc * scale[None, :] + b_ref[...][None, :], 0.0)


_finish = pl.pallas_call(
    _finish_body,
    out_shape=jax.ShapeDtypeStruct((N, D), jnp.float32),
)


def _edge_body(asrc_hbm, adst_hbm, srcf_hbm, dstf_hbm,
               eexp_hbm, denp_hbm,
               asrc_v, adst_v, src_v, dst_v, eexp_v, denom_v):
    cid = lax.axis_index("c")
    sid = lax.axis_index("s")
    wid = sid * NC + cid

    zeros16 = jnp.zeros((L,), jnp.float32)

    @pl.loop(0, NP // L)
    def _zero_denom(i):
        denom_v[pl.ds(i * L, L)] = zeros16

    pltpu.sync_copy(asrc_hbm, asrc_v)
    pltpu.sync_copy(adst_hbm, adst_v)
    pltpu.sync_copy(srcf_hbm.at[pl.ds(wid * EPT, EPT)], src_v)
    pltpu.sync_copy(dstf_hbm.at[pl.ds(wid * EPT, EPT)], dst_v)

    @plsc.parallel_loop(0, NCH, unroll=2)
    def _edge_chunk(j):
        for v in range(K // L):
            o = j * K + v * L
            sidx = src_v[pl.ds(o, L)]
            didx = dst_v[pl.ds(o, L)]
            av = plsc.load_gather(asrc_v, [sidx])
            bv = plsc.load_gather(adst_v, [didx])
            e = av + bv
            e = jnp.where(e >= 0.0, e, 0.2 * e)
            ex = jnp.exp(e)
            eexp_v[pl.ds(o, L)] = ex
            plsc.addupdate_scatter(denom_v, [didx], ex)

    pltpu.sync_copy(eexp_v, eexp_hbm.at[pl.ds(wid * EPT, EPT)])
    pltpu.sync_copy(denom_v, denp_hbm.at[wid])


_edge_sc = functools.partial(
    pl.kernel,
    out_type=[
        jax.ShapeDtypeStruct((E,), jnp.float32),
        jax.ShapeDtypeStruct((NT, NP), jnp.float32),
    ],
    mesh=plsc.VectorSubcoreMesh(core_axis_name="c", subcore_axis_name="s"),
    compiler_params=pltpu.CompilerParams(needs_layout_passes=False),
    scratch_types=[
        pltpu.VMEM((N,), jnp.float32),
        pltpu.VMEM((N,), jnp.float32),
        pltpu.VMEM((EPT,), jnp.int32),
        pltpu.VMEM((EPT,), jnp.int32),
        pltpu.VMEM((EPT,), jnp.float32),
        pltpu.VMEM((NP,), jnp.float32),
    ],
)(_edge_body)


def _rows_body(h_hbm, eexpf_hbm, srcf_hbm, dst4_hbm,
               outp_hbm,
               src_blk0, src_blk1, dst_blk0, dst_blk1, eexp_blk0, eexp_blk1,
               rows_0, rows_1, rows_2, rows_3,
               acc_sh,
               gsem0, gsem1, gsem2, gsem3, ssem0, ssem1, ssem2, ssem3,
               stsem0, stsem1):
    cid = lax.axis_index("c")
    sid = lax.axis_index("s")
    wid = sid * NC + cid
    src_blks = (src_blk0, src_blk1)
    dst_blks = (dst_blk0, dst_blk1)
    eexp_blks = (eexp_blk0, eexp_blk1)
    rowss = (rows_0, rows_1, rows_2, rows_3)
    gsem = (gsem0, gsem1, gsem2, gsem3)
    ssem = (ssem0, ssem1, ssem2, ssem3)
    stsem = (stsem0, stsem1)

    zeros16 = jnp.zeros((L,), jnp.float32)

    @pl.loop(0, K)
    def _zero_rows(r):
        for c in range(D // L):
            rows_0[r, pl.ds(c * L, L)] = zeros16

    base_r = sid * RPT
    for t in range(RPT // K):
        pltpu.sync_copy(rows_0, acc_sh.at[pl.ds(base_r + t * K, K)])

    pltpu.sync_copy(srcf_hbm.at[pl.ds(wid * EPT, B * K)], src_blk0)
    pltpu.sync_copy(dst4_hbm.at[wid, 0], dst_blk0)
    pltpu.sync_copy(eexpf_hbm.at[pl.ds(wid * EPT, B * K)], eexp_blk0)

    plsc.subcore_barrier()

    pltpu.async_copy(h_hbm.at[src_blk0.at[pl.ds(0, K)]], rows_0, gsem[0])

    def wait_gather(p):
        pltpu.make_async_copy(h_hbm.at[src_blk0.at[pl.ds(0, K)]],
                              rowss[p], gsem[p]).wait()

    def wait_scatter(p):
        pltpu.make_async_copy(rowss[p],
                              acc_sh.at[dst_blk0.at[0]], ssem[p]).wait()

    def stage_issue(bi_next, nb):
        pltpu.async_copy(
            srcf_hbm.at[pl.ds(wid * EPT + bi_next * (B * K), B * K)],
            src_blks[nb], stsem[nb])
        pltpu.async_copy(dst4_hbm.at[wid, bi_next], dst_blks[nb], stsem[nb])
        pltpu.async_copy(
            eexpf_hbm.at[pl.ds(wid * EPT + bi_next * (B * K), B * K)],
            eexp_blks[nb], stsem[nb])

    def wait_stage(nb):
        pltpu.make_async_copy(srcf_hbm.at[pl.ds(0, B * K)], src_blks[nb],
                              stsem[nb]).wait()
        pltpu.make_async_copy(dst4_hbm.at[wid, 0], dst_blks[nb],
                              stsem[nb]).wait()
        pltpu.make_async_copy(eexpf_hbm.at[pl.ds(0, B * K)], eexp_blks[nb],
                              stsem[nb]).wait()

    def chunk(pb, b, p, *, skip_scatter_wait=False, stage_bi=None,
              next_rc=None):
        q = (p + 1) % 4
        if stage_bi is not None:
            stage_issue(stage_bi, 1 - pb)
        if not skip_scatter_wait:
            wait_scatter(q)
        if next_rc is not None:
            npb, nb = next_rc
            if b == B - 1:
                wait_stage(npb)
            pltpu.async_copy(h_hbm.at[src_blks[npb].at[pl.ds(nb * K, K)]],
                             rowss[q], gsem[q])
        wait_gather(p)

        rows_p = rowss[p]
        eexp_pb = eexp_blks[pb]

        @pl.loop(0, K // L)
        def _scale_grp(g):
            w16 = eexp_pb[pl.ds(b * K + g * L, L)]

            @plsc.parallel_loop(0, L, unroll=4)
            def _scale_row(rr):
                w = w16.at[jnp.full((L,), rr, dtype=jnp.int32)].get(
                    mode="promise_in_bounds")
                r = g * L + rr
                for c in range(D // L):
                    rows_p[r, pl.ds(c * L, L)] = rows_p[r, pl.ds(c * L, L)] * w

        pltpu.async_copy(rows_p, acc_sh.at[dst_blks[pb].at[b]],
                         ssem[p], add=True)

    def group(gbase, *, first=False, last=False):
        nblk_here = 1 if last else GRP
        for bb in range(nblk_here):
            pb = bb % 2
            for b in range(B):
                t_in_group = bb * B + b
                chunk(
                    pb, b, t_in_group % 4,
                    skip_scatter_wait=first and t_in_group < 3,
                    stage_bi=(gbase + bb + 1) if (b == 2 and not last) else None,
                    next_rc=None if (last and b == B - 1) else
                            ((pb, b + 1) if b < B - 1 else (1 - pb, 0)),
                )

    group(jnp.int32(0), first=True)

    @pl.loop(1, (NBLK - 1) // GRP)
    def _grp(g):
        group(g * GRP)

    group(jnp.int32(NBLK - 1), last=True)

    wait_scatter((NCH - 3) % 4)
    wait_scatter((NCH - 2) % 4)
    wait_scatter((NCH - 1) % 4)
    plsc.subcore_barrier()
    pltpu.sync_copy(acc_sh.at[pl.ds(base_r, RPT)],
                    outp_hbm.at[cid, pl.ds(base_r, RPT)])


_rows_sc = functools.partial(
    pl.kernel,
    out_type=jax.ShapeDtypeStruct((NC, NP, D), jnp.float32),
    mesh=plsc.VectorSubcoreMesh(core_axis_name="c", subcore_axis_name="s"),
    compiler_params=pltpu.CompilerParams(needs_layout_passes=False),
    scratch_types=[
        pltpu.VMEM((B * K,), jnp.int32),
        pltpu.VMEM((B * K,), jnp.int32),
        pltpu.VMEM((B, K), jnp.int32),
        pltpu.VMEM((B, K), jnp.int32),
        pltpu.VMEM((B * K,), jnp.float32),
        pltpu.VMEM((B * K,), jnp.float32),
        pltpu.VMEM((K, D), jnp.float32),
        pltpu.VMEM((K, D), jnp.float32),
        pltpu.VMEM((K, D), jnp.float32),
        pltpu.VMEM((K, D), jnp.float32),
        pltpu.VMEM_SHARED((NP, D), jnp.float32),
        pltpu.SemaphoreType.DMA,
        pltpu.SemaphoreType.DMA,
        pltpu.SemaphoreType.DMA,
        pltpu.SemaphoreType.DMA,
        pltpu.SemaphoreType.DMA,
        pltpu.SemaphoreType.DMA,
        pltpu.SemaphoreType.DMA,
        pltpu.SemaphoreType.DMA,
        pltpu.SemaphoreType.DMA,
        pltpu.SemaphoreType.DMA,
    ],
)(_rows_body)


def kernel(x, adj, W, att_src, att_dst, bn_gamma, bn_beta):
    src_f = adj[0].astype(jnp.int32)
    dst_f = adj[1].astype(jnp.int32)
    dst4 = dst_f.reshape(NT, NBLK, B, K)
    h, a_src, a_dst = _prep(x, W, att_src, att_dst)
    eexp_f, denp = _edge_sc(a_src, a_dst, src_f, dst_f)
    outp = _rows_sc(h, eexp_f, src_f, dst4)
    return _finish(outp, denp, bn_gamma, bn_beta)
